# Initial kernel scaffold; baseline (speedup 1.0000x reference)
import jax
import jax.numpy as jnp
from jax.experimental import pallas as pl


def kernel(x, edge_index, W1, bn_gamma, bn_beta, W2, b2, Wx1, bx1, Wx2, Wm, bm):
    raise NotImplementedError("write your pallas kernel here")


# speedup vs baseline: 4.2854x; 4.2854x over previous
import functools

import jax
import jax.numpy as jnp
from jax import lax
from jax.experimental import pallas as pl
from jax.experimental.pallas import tpu as pltpu
from jax.experimental.pallas import tpu_sc as plsc

N = 50000
E = 1600000
D = 3
H = 32
BN_EPS = 1e-5

XW = 16
AW = 8
CH = 128
NCH = E // CH
NW = 32
BASE_T = NCH // NW
EXTRA_T = NCH % NW
NP = 51200
PB = NP // 16

_G16 = CH // 16


def _psi(v):
    return jnp.sign(v) * jnp.log(jnp.abs(v) + 1.0)


def _sc_a_body(x0, x1, x2, i_hbm, j_hbm, nrm_o, dot_o, dx_o, dy_o, dz_o,
               ii_v, jj_v, bi0, bi1, bi2, bj0, bj1, bj2,
               st_n, st_d, st_x, st_y, st_z, sem):
    wid = lax.axis_index("s") * 2 + lax.axis_index("c")
    nt = BASE_T + jnp.where(wid < EXTRA_T, 1, 0)

    def chunk(t, carry):
        off = (wid + t * NW) * CH
        pltpu.sync_copy(i_hbm.at[pl.ds(off, CH)], ii_v)
        pltpu.sync_copy(j_hbm.at[pl.ds(off, CH)], jj_v)
        cps = [pltpu.async_copy(x0.at[ii_v], bi0, sem),
               pltpu.async_copy(x1.at[ii_v], bi1, sem),
               pltpu.async_copy(x2.at[ii_v], bi2, sem),
               pltpu.async_copy(x0.at[jj_v], bj0, sem),
               pltpu.async_copy(x1.at[jj_v], bj1, sem),
               pltpu.async_copy(x2.at[jj_v], bj2, sem)]
        for cp in cps:
            cp.wait()
        for g in range(_G16):
            s = pl.ds(g * 16, 16)
            xi0, xi1, xi2 = bi0[s], bi1[s], bi2[s]
            xj0, xj1, xj2 = bj0[s], bj1[s], bj2[s]
            dx = xi0 - xj0
            dy = xi1 - xj1
            dz = xi2 - xj2
            st_x[s] = dx
            st_y[s] = dy
            st_z[s] = dz
            st_n[s] = dx * dx + dy * dy + dz * dz
            st_d[s] = xi0 * xj0 + xi1 * xj1 + xi2 * xj2
        pltpu.sync_copy(st_n, nrm_o.at[pl.ds(off, CH)])
        pltpu.sync_copy(st_d, dot_o.at[pl.ds(off, CH)])
        pltpu.sync_copy(st_x, dx_o.at[pl.ds(off, CH)])
        pltpu.sync_copy(st_y, dy_o.at[pl.ds(off, CH)])
        pltpu.sync_copy(st_z, dz_o.at[pl.ds(off, CH)])
        return carry

    lax.fori_loop(0, nt, chunk, 0)


def _sc_a(x0, x1, x2, i_idx, j_idx):
    f32 = jnp.float32
    out = jax.ShapeDtypeStruct((E,), f32)
    mesh = plsc.VectorSubcoreMesh(core_axis_name="c", subcore_axis_name="s")
    return pl.kernel(
        _sc_a_body,
        out_type=[out] * 5,
        mesh=mesh,
        scratch_types=(
            [pltpu.VMEM((CH,), jnp.int32)] * 2
            + [pltpu.VMEM((CH,), f32)] * 11
            + [pltpu.SemaphoreType.DMA]
        ),
    )(x0, x1, x2, i_idx, j_idx)


def _tc1_body(nr_ref, dr_ref, n_out, d_out, s_out):
    b = pl.program_id(0)
    n = jnp.log(nr_ref[0] + 1.0)
    d = _psi(dr_ref[0])
    n_out[0] = n
    d_out[0] = d

    @pl.when(b == 0)
    def _():
        for k in range(8):
            s_out[k] = 0.0

    s_out[0] += jnp.sum(n)
    s_out[1] += jnp.sum(d)
    s_out[2] += jnp.sum(n * n)
    s_out[3] += jnp.sum(d * d)
    s_out[4] += jnp.sum(n * d)


def _tc1(nrm, dot):
    f32 = jnp.float32
    R = 500
    nb = (E // 128) // R
    grid = (nb,)
    blk = pl.BlockSpec((1, R, 128), lambda b: (b, 0, 0))
    return pl.pallas_call(
        _tc1_body,
        grid=grid,
        in_specs=[blk, blk],
        out_specs=[blk, blk,
                   pl.BlockSpec(memory_space=pltpu.SMEM)],
        out_shape=[jax.ShapeDtypeStruct((nb, R, 128), f32),
                   jax.ShapeDtypeStruct((nb, R, 128), f32),
                   jax.ShapeDtypeStruct((8,), f32)],
    )(nrm.reshape(nb, R, 128), dot.reshape(nb, R, 128))


_RB = 1000


def _tc2_body(n4_ref, d4_ref, mats_ref, cols_ref, r4_ref, vecs_ref,
              m2_ref, phi4_ref):
    f32 = jnp.float32
    r4 = r4_ref[...]
    nr = jnp.dot(n4_ref[...], r4, preferred_element_type=f32)
    dr = jnp.dot(d4_ref[...], r4, preferred_element_type=f32)
    u = vecs_ref[0:1, :]
    v = vecs_ref[1:2, :]
    w = vecs_ref[2:3, :]
    h = jnp.maximum(nr * u + dr * v + w, 0.0)
    m1 = jnp.dot(h, mats_ref[0:128, :], preferred_element_type=f32)
    m1 = jnp.maximum(m1 + vecs_ref[3:4, :], 0.0)
    g4 = jnp.dot(m1 * vecs_ref[6:7, :], cols_ref[:, 4:8],
                 preferred_element_type=f32)
    g4 = 1.0 / (1.0 + jnp.exp(-(g4 + vecs_ref[5:6, 0:4])))
    gate = jnp.dot(g4, r4, preferred_element_type=f32)
    mij = m1 * gate
    t = jnp.dot(mij, mats_ref[128:256, :], preferred_element_type=f32)
    t = jnp.maximum(t + vecs_ref[4:5, :], 0.0)
    phi4 = jnp.dot(t, cols_ref[:, 0:4], preferred_element_type=f32)
    m2_ref[...] = mij
    phi4_ref[...] = phi4


def _tc2(n4, d4, mats, cols, r4, vecs):
    f32 = jnp.float32
    E4 = E // 4
    grid = (E4 // _RB,)
    full = lambda shp: pl.BlockSpec(shp, lambda b: tuple(0 for _ in shp))
    return pl.pallas_call(
        _tc2_body,
        grid=grid,
        in_specs=[pl.BlockSpec((_RB, 4), lambda b: (b, 0)),
                  pl.BlockSpec((_RB, 4), lambda b: (b, 0)),
                  full((256, 128)), full((128, 8)), full((4, 128)),
                  full((8, 128))],
        out_specs=[pl.BlockSpec((_RB, 128), lambda b: (b, 0)),
                   pl.BlockSpec((_RB, 4), lambda b: (b, 0))],
        out_shape=[jax.ShapeDtypeStruct((E4, 128), f32),
                   jax.ShapeDtypeStruct((E4, 4), f32)],
    )(n4, d4, mats, cols, r4, vecs)


def _sc_b_body(dx_p, dy_p, dz_p, phi_p, i_hbm, zrows,
               p0x, p0y, p0z, p0c, p1x, p1y, p1z, p1c,
               ii_v, bx, by, bz, bp, vx_b, vy_b, vz_b, on_b,
               accx, accy, accz, accc):
    cid = lax.axis_index("c")
    sid = lax.axis_index("s")
    wid = sid * 2 + cid
    nt = BASE_T + jnp.where(wid < EXTRA_T, 1, 0)

    for g in range(_G16):
        on_b[pl.ds(g * 16, 16)] = jnp.ones((16,), jnp.float32)
    zs = pl.ds(sid * PB, PB)
    pltpu.sync_copy(zrows.at[zs], accx.at[zs])
    pltpu.sync_copy(zrows.at[zs], accy.at[zs])
    pltpu.sync_copy(zrows.at[zs], accz.at[zs])
    pltpu.sync_copy(zrows.at[zs], accc.at[zs])
    plsc.subcore_barrier()

    def chunk(t, carry):
        off = (wid + t * NW) * CH
        pltpu.sync_copy(i_hbm.at[pl.ds(off, CH)], ii_v)
        pltpu.sync_copy(dx_p.at[pl.ds(off, CH)], bx)
        pltpu.sync_copy(dy_p.at[pl.ds(off, CH)], by)
        pltpu.sync_copy(dz_p.at[pl.ds(off, CH)], bz)
        pltpu.sync_copy(phi_p.at[pl.ds(off, CH)], bp)
        for g in range(_G16):
            s = pl.ds(g * 16, 16)
            ph = bp[s]
            vx_b[s] = jnp.clip(bx[s] * ph, -100.0, 100.0)
            vy_b[s] = jnp.clip(by[s] * ph, -100.0, 100.0)
            vz_b[s] = jnp.clip(bz[s] * ph, -100.0, 100.0)
        pltpu.sync_copy(vx_b, accx.at[ii_v], add=True)
        pltpu.sync_copy(vy_b, accy.at[ii_v], add=True)
        pltpu.sync_copy(vz_b, accz.at[ii_v], add=True)
        pltpu.sync_copy(on_b, accc.at[ii_v], add=True)
        return carry

    lax.fori_loop(0, nt, chunk, 0)
    plsc.subcore_barrier()

    @pl.when(cid == 0)
    def _():
        pltpu.sync_copy(accx.at[zs], p0x.at[zs])
        pltpu.sync_copy(accy.at[zs], p0y.at[zs])
        pltpu.sync_copy(accz.at[zs], p0z.at[zs])
        pltpu.sync_copy(accc.at[zs], p0c.at[zs])

    @pl.when(cid == 1)
    def _():
        pltpu.sync_copy(accx.at[zs], p1x.at[zs])
        pltpu.sync_copy(accy.at[zs], p1y.at[zs])
        pltpu.sync_copy(accz.at[zs], p1z.at[zs])
        pltpu.sync_copy(accc.at[zs], p1c.at[zs])


def _sc_b(dx_p, dy_p, dz_p, phi_p, i_idx, zrows):
    f32 = jnp.float32
    mesh = plsc.VectorSubcoreMesh(core_axis_name="c", subcore_axis_name="s")
    return pl.kernel(
        _sc_b_body,
        out_type=[jax.ShapeDtypeStruct((NP,), f32)] * 8,
        mesh=mesh,
        scratch_types=(
            [pltpu.VMEM((CH,), jnp.int32)]
            + [pltpu.VMEM((CH,), f32)] * 8
            + [pltpu.VMEM_SHARED((NP,), f32)] * 4
        ),
    )(dx_p, dy_p, dz_p, phi_p, i_idx, zrows)


def _tc3_body(xp_ref, p_ref, o_ref):
    p = p_ref[0]
    s = p[0:4, :] + p[4:8, :]
    cnt = jnp.maximum(s[3:4, :], 1.0)
    o_ref[0] = xp_ref[0] + s[0:3, :] / cnt


def _tc3(xp, pr):
    f32 = jnp.float32
    NB = 2000
    nb = N // NB
    return pl.pallas_call(
        _tc3_body,
        grid=(nb,),
        in_specs=[pl.BlockSpec((1, 3, NB), lambda b: (b, 0, 0)),
                  pl.BlockSpec((1, 8, NB), lambda b: (b, 0, 0))],
        out_specs=pl.BlockSpec((1, 3, NB), lambda b: (b, 0, 0)),
        out_shape=jax.ShapeDtypeStruct((nb, 3, NB), f32),
    )(xp, pr)


def kernel(x, edge_index, W1, bn_gamma, bn_beta, W2, b2, Wx1, bx1, Wx2, Wm, bm):
    f32 = jnp.float32
    i_idx = edge_index[0]
    j_idx = edge_index[1]
    xt = x.T
    nrm_raw, dot_raw, dxp, dyp, dzp = _sc_a(xt[0], xt[1], xt[2], i_idx, j_idx)

    n_psi, d_psi, sums = _tc1(nrm_raw, dot_raw)

    En, Ed, En2, Ed2, End = (sums[k] / E for k in range(5))
    a = W1[:, 0]
    b_ = W1[:, 1]
    mu = a * En + b_ * Ed
    eh2 = a * a * En2 + 2.0 * a * b_ * End + b_ * b_ * Ed2
    var = eh2 - mu * mu
    sc = bn_gamma * lax.rsqrt(var + BN_EPS)
    u = a * sc
    v = b_ * sc
    w0 = bn_beta - mu * sc

    eye4 = jnp.eye(4, dtype=f32)
    ones32r = jnp.ones((1, 32), f32)
    tile4 = lambda vec: jnp.tile(vec.reshape(1, H), (1, 4)).reshape(1, 128)
    vecs = jnp.concatenate([
        tile4(u), tile4(v), tile4(w0), tile4(b2), tile4(bx1),
        jnp.full((1, 128), bm[0], f32), tile4(Wm[0]),
        jnp.zeros((1, 128), f32),
    ], axis=0)
    mats = jnp.concatenate([
        jnp.kron(eye4, W2.T), jnp.kron(eye4, Wx1.T)], axis=0)
    e0 = jnp.zeros((H, 1), f32).at[0, 0].set(1.0)
    cols = jnp.concatenate([
        jnp.kron(eye4, Wx2.T),
        jnp.kron(eye4, jnp.ones((H, 1), f32)),
    ], axis=1)
    del e0
    r4 = jnp.kron(eye4, ones32r)

    n4 = n_psi.reshape(E // 4, 4)
    d4 = d_psi.reshape(E // 4, 4)
    m2, phi4 = _tc2(n4, d4, mats, cols, r4, vecs)
    m_ij = m2.reshape(E, H)
    phi = phi4.reshape(E)

    zrows = jnp.zeros((NP,), f32)
    planes = _sc_b(dxp, dyp, dzp, phi, i_idx, zrows)

    NB = 2000
    pr = jnp.stack([p[:N] for p in planes])
    pr = pr.reshape(8, N // NB, NB).swapaxes(0, 1)
    xp = xt.reshape(3, N // NB, NB).swapaxes(0, 1)
    xt_t = _tc3(xp, pr)
    x_tilde = xt_t.swapaxes(0, 1).reshape(3, N).T
    return (x_tilde, m_ij)


# speedup vs baseline: 5.9254x; 1.3827x over previous
import functools

import jax
import jax.numpy as jnp
from jax import lax
from jax.experimental import pallas as pl
from jax.experimental.pallas import tpu as pltpu
from jax.experimental.pallas import tpu_sc as plsc

N = 50000
E = 1600000
D = 3
H = 32
BN_EPS = 1e-5

CH = 128
NCH = E // CH
NW = 32
BASE_T = NCH // NW
EXTRA_T = NCH % NW
NT2 = (BASE_T + 1) // 2 + 2
NPAD = 51200
PB = NPAD // 16

_G16 = CH // 16


def _psi(v):
    return jnp.sign(v) * jnp.log(jnp.abs(v) + 1.0)


def _sc_a_body(x0, x1, x2, i_hbm, j_hbm, nrm_o, dot_o, dx_o, dy_o, dz_o, *sc):
    wid = lax.axis_index("s") * 2 + lax.axis_index("c")
    nt = BASE_T + jnp.where(wid < EXTRA_T, 1, 0)
    ii = sc[0:2]
    jj = sc[2:4]
    gi = (sc[4:7], sc[7:10])
    gj = (sc[10:13], sc[13:16])
    st = (sc[16:21], sc[21:26])
    s_idx = sc[26:28]
    s_gat = sc[28:30]
    s_out = sc[30:32]
    tabs = (x0, x1, x2)
    outs = (nrm_o, dot_o, dx_o, dy_o, dz_o)

    def issue_gat(s):
        for k in range(3):
            pltpu.async_copy(tabs[k].at[ii[s]], gi[s][k], s_gat[s])
            pltpu.async_copy(tabs[k].at[jj[s]], gj[s][k], s_gat[s])

    def drain_gat(s):
        for k in range(3):
            pltpu.make_async_copy(x0.at[pl.ds(0, CH)], gi[s][k], s_gat[s]).wait()
            pltpu.make_async_copy(x0.at[pl.ds(0, CH)], gj[s][k], s_gat[s]).wait()

    def drain_out(s):
        for k in range(5):
            pltpu.make_async_copy(st[s][k], outs[k].at[pl.ds(0, CH)],
                                  s_out[s]).wait()

    def compute(s, off):
        for g in range(_G16):
            sl = pl.ds(g * 16, 16)
            a0, a1, a2 = gi[s][0][sl], gi[s][1][sl], gi[s][2][sl]
            b0, b1, b2 = gj[s][0][sl], gj[s][1][sl], gj[s][2][sl]
            dx = a0 - b0
            dy = a1 - b1
            dz = a2 - b2
            st[s][2][sl] = dx
            st[s][3][sl] = dy
            st[s][4][sl] = dz
            st[s][0][sl] = dx * dx + dy * dy + dz * dz
            st[s][1][sl] = a0 * b0 + a1 * b1 + a2 * b2
        for k in range(5):
            pltpu.async_copy(st[s][k], outs[k].at[pl.ds(off, CH)], s_out[s])

    off0 = wid * CH
    pltpu.sync_copy(i_hbm.at[pl.ds(off0, CH)], ii[0])
    pltpu.sync_copy(j_hbm.at[pl.ds(off0, CH)], jj[0])
    issue_gat(0)
    off1 = (wid + NW) * CH
    pltpu.async_copy(i_hbm.at[pl.ds(off1, CH)], ii[1], s_idx[1])
    pltpu.async_copy(j_hbm.at[pl.ds(off1, CH)], jj[1], s_idx[1])

    def it(t2, carry):
        for b in (0, 1):
            t = t2 * 2 + b

            @pl.when(t < nt)
            def _():
                drain_gat(b)

            @pl.when(jnp.logical_and(t >= 2, t - 2 < nt))
            def _():
                drain_out(b)

            @pl.when(t + 1 < nt)
            def _():
                pltpu.make_async_copy(i_hbm.at[pl.ds(0, CH)], ii[1 - b],
                                      s_idx[1 - b]).wait()
                pltpu.make_async_copy(i_hbm.at[pl.ds(0, CH)], jj[1 - b],
                                      s_idx[1 - b]).wait()
                issue_gat(1 - b)

            @pl.when(t + 2 < nt)
            def _():
                offn = (wid + (t + 2) * NW) * CH
                pltpu.async_copy(i_hbm.at[pl.ds(offn, CH)], ii[b], s_idx[b])
                pltpu.async_copy(j_hbm.at[pl.ds(offn, CH)], jj[b], s_idx[b])

            @pl.when(t < nt)
            def _():
                compute(b, (wid + t * NW) * CH)
        return carry

    lax.fori_loop(0, NT2, it, 0)


def _sc_a(x0, x1, x2, i_idx, j_idx):
    f32 = jnp.float32
    out = jax.ShapeDtypeStruct((E,), f32)
    mesh = plsc.VectorSubcoreMesh(core_axis_name="c", subcore_axis_name="s")
    return pl.kernel(
        _sc_a_body,
        out_type=[out] * 5,
        mesh=mesh,
        scratch_types=(
            [pltpu.VMEM((CH,), jnp.int32)] * 4
            + [pltpu.VMEM((CH,), f32)] * 22
            + [pltpu.SemaphoreType.DMA] * 6
        ),
    )(x0, x1, x2, i_idx, j_idx)


def _tc1_body(nr_ref, dr_ref, n_out, d_out, s_out):
    b = pl.program_id(0)
    n = jnp.log(nr_ref[0] + 1.0)
    d = _psi(dr_ref[0])
    n_out[0] = n
    d_out[0] = d

    @pl.when(b == 0)
    def _():
        for k in range(8):
            s_out[k] = 0.0

    s_out[0] += jnp.sum(n)
    s_out[1] += jnp.sum(d)
    s_out[2] += jnp.sum(n * n)
    s_out[3] += jnp.sum(d * d)
    s_out[4] += jnp.sum(n * d)


def _tc1(nrm, dot):
    f32 = jnp.float32
    R = 500
    nb = (E // 128) // R
    grid = (nb,)
    blk = pl.BlockSpec((1, R, 128), lambda b: (b, 0, 0))
    return pl.pallas_call(
        _tc1_body,
        grid=grid,
        in_specs=[blk, blk],
        out_specs=[blk, blk,
                   pl.BlockSpec(memory_space=pltpu.SMEM)],
        out_shape=[jax.ShapeDtypeStruct((nb, R, 128), f32),
                   jax.ShapeDtypeStruct((nb, R, 128), f32),
                   jax.ShapeDtypeStruct((8,), f32)],
    )(nrm.reshape(nb, R, 128), dot.reshape(nb, R, 128))


_RB = 1000


def _tc2_body(n4_ref, d4_ref, mats_ref, cols_ref, r4_ref, vecs_ref,
              m2_ref, phi4_ref):
    f32 = jnp.float32
    r4 = r4_ref[...]
    nr = jnp.dot(n4_ref[...], r4, preferred_element_type=f32)
    dr = jnp.dot(d4_ref[...], r4, preferred_element_type=f32)
    u = vecs_ref[0:1, :]
    v = vecs_ref[1:2, :]
    w = vecs_ref[2:3, :]
    h = jnp.maximum(nr * u + dr * v + w, 0.0)
    m1 = jnp.dot(h, mats_ref[0:128, :], preferred_element_type=f32)
    m1 = jnp.maximum(m1 + vecs_ref[3:4, :], 0.0)
    g4 = jnp.dot(m1 * vecs_ref[6:7, :], cols_ref[:, 4:8],
                 preferred_element_type=f32)
    g4 = 1.0 / (1.0 + jnp.exp(-(g4 + vecs_ref[5:6, 0:4])))
    gate = jnp.dot(g4, r4, preferred_element_type=f32)
    mij = m1 * gate
    t = jnp.dot(mij, mats_ref[128:256, :], preferred_element_type=f32)
    t = jnp.maximum(t + vecs_ref[4:5, :], 0.0)
    phi4 = jnp.dot(t, cols_ref[:, 0:4], preferred_element_type=f32)
    m2_ref[...] = mij
    phi4_ref[...] = phi4


def _tc2(n4, d4, mats, cols, r4, vecs):
    f32 = jnp.float32
    E4 = E // 4
    grid = (E4 // _RB,)
    full = lambda shp: pl.BlockSpec(shp, lambda b: tuple(0 for _ in shp))
    return pl.pallas_call(
        _tc2_body,
        grid=grid,
        in_specs=[pl.BlockSpec((_RB, 4), lambda b: (b, 0)),
                  pl.BlockSpec((_RB, 4), lambda b: (b, 0)),
                  full((256, 128)), full((128, 8)), full((4, 128)),
                  full((8, 128))],
        out_specs=[pl.BlockSpec((_RB, 128), lambda b: (b, 0)),
                   pl.BlockSpec((_RB, 4), lambda b: (b, 0))],
        out_shape=[jax.ShapeDtypeStruct((E4, 128), f32),
                   jax.ShapeDtypeStruct((E4, 4), f32)],
    )(n4, d4, mats, cols, r4, vecs)


def _sc_b_body(dx_p, dy_p, dz_p, phi_p, i_hbm,
               p0x, p0y, p0z, p0c, p1x, p1y, p1z, p1c, *sc):
    cid = lax.axis_index("c")
    sid = lax.axis_index("s")
    wid = sid * 2 + cid
    nt = BASE_T + jnp.where(wid < EXTRA_T, 1, 0)
    f32 = jnp.float32
    ld = (sc[0:5], sc[5:10])
    on_b = sc[10]
    zbuf = sc[11]
    accs = sc[12:16]
    s_ld = sc[16:18]
    srcs = (i_hbm, dx_p, dy_p, dz_p, phi_p)
    pls = ((p0x, p0y, p0z, p0c), (p1x, p1y, p1z, p1c))

    for g in range(_G16):
        on_b[pl.ds(g * 16, 16)] = jnp.ones((16,), f32)

    def zst(k, c):
        zbuf[pl.ds(k * 16, 16)] = jnp.zeros((16,), f32)
        return c

    lax.fori_loop(0, PB // 16, zst, 0)

    for a in accs:
        pltpu.sync_copy(zbuf, a.at[pl.ds(sid * PB, PB)])

    plsc.subcore_barrier()

    for sref, dbuf in zip(srcs, ld[0]):
        pltpu.sync_copy(sref.at[pl.ds(wid * CH, CH)], dbuf)
    for sref, dbuf in zip(srcs, ld[1]):
        pltpu.async_copy(sref.at[pl.ds((wid + NW) * CH, CH)], dbuf, s_ld[1])

    def it(t2, carry):
        for b in (0, 1):
            t = t2 * 2 + b

            @pl.when(jnp.logical_and(t >= 1, t < nt))
            def _():
                for sref, dbuf in zip(srcs, ld[b]):
                    pltpu.make_async_copy(sref.at[pl.ds(0, CH)], dbuf,
                                          s_ld[b]).wait()

            @pl.when(t < nt)
            def _():
                iv, bx, by, bz, bp = ld[b]
                for g in range(_G16):
                    sl = pl.ds(g * 16, 16)
                    ph = bp[sl]
                    bx[sl] = jnp.clip(bx[sl] * ph, -100.0, 100.0)
                    by[sl] = jnp.clip(by[sl] * ph, -100.0, 100.0)
                    bz[sl] = jnp.clip(bz[sl] * ph, -100.0, 100.0)
                pltpu.sync_copy(bx, accs[0].at[iv], add=True)
                pltpu.sync_copy(by, accs[1].at[iv], add=True)
                pltpu.sync_copy(bz, accs[2].at[iv], add=True)
                pltpu.sync_copy(on_b, accs[3].at[iv], add=True)

            @pl.when(t + 2 < nt)
            def _():
                offn = (wid + (t + 2) * NW) * CH
                for sref, dbuf in zip(srcs, ld[b]):
                    pltpu.async_copy(sref.at[pl.ds(offn, CH)], dbuf, s_ld[b])
        return carry

    lax.fori_loop(0, NT2, it, 0)
    plsc.subcore_barrier()

    for ci in (0, 1):
        @pl.when(cid == ci)
        def _(ci=ci):
            for a, p in zip(accs, pls[ci]):
                pltpu.sync_copy(a.at[pl.ds(sid * PB, PB)],
                                p.at[pl.ds(sid * PB, PB)])


def _sc_b(dx_p, dy_p, dz_p, phi_p, i_idx):
    f32 = jnp.float32
    mesh = plsc.VectorSubcoreMesh(core_axis_name="c", subcore_axis_name="s")
    return pl.kernel(
        _sc_b_body,
        out_type=[jax.ShapeDtypeStruct((NPAD,), f32)] * 8,
        mesh=mesh,
        scratch_types=(
            ([pltpu.VMEM((CH,), jnp.int32)] + [pltpu.VMEM((CH,), f32)] * 4) * 2
            + [pltpu.VMEM((CH,), f32)]
            + [pltpu.VMEM((PB,), f32)]
            + [pltpu.VMEM_SHARED((NPAD,), f32)] * 4
            + [pltpu.SemaphoreType.DMA] * 2
        ),
    )(dx_p, dy_p, dz_p, phi_p, i_idx)


def _tc3_body(x3_ref, p0x, p0y, p0z, p0c, p1x, p1y, p1z, p1c, o_ref):
    cnt = jnp.maximum(p0c[0] + p1c[0], 1.0)
    ux = (p0x[0] + p1x[0]) / cnt
    uy = (p0y[0] + p1y[0]) / cnt
    uz = (p0z[0] + p1z[0]) / cnt
    upd = jnp.concatenate([ux, uy, uz], axis=0)
    o_ref[0] = x3_ref[0] + upd.T


def _tc3(x3, planes):
    f32 = jnp.float32
    NB = 2048
    nb = NPAD // NB
    pblk = pl.BlockSpec((1, 1, NB), lambda b: (b, 0, 0))
    return pl.pallas_call(
        _tc3_body,
        grid=(nb,),
        in_specs=[pl.BlockSpec((1, NB, 3), lambda b: (b, 0, 0))] +
                 [pblk] * 8,
        out_specs=pl.BlockSpec((1, NB, 3), lambda b: (b, 0, 0)),
        out_shape=jax.ShapeDtypeStruct((nb, NB, 3), f32),
    )(x3, *planes)


def kernel(x, edge_index, W1, bn_gamma, bn_beta, W2, b2, Wx1, bx1, Wx2, Wm, bm):
    f32 = jnp.float32
    i_idx = edge_index[0]
    j_idx = edge_index[1]
    xt = x.T
    nrm_raw, dot_raw, dxp, dyp, dzp = _sc_a(xt[0], xt[1], xt[2], i_idx, j_idx)

    n_psi, d_psi, sums = _tc1(nrm_raw, dot_raw)

    En, Ed, En2, Ed2, End = (sums[k] / E for k in range(5))
    a = W1[:, 0]
    b_ = W1[:, 1]
    mu = a * En + b_ * Ed
    eh2 = a * a * En2 + 2.0 * a * b_ * End + b_ * b_ * Ed2
    var = eh2 - mu * mu
    sc = bn_gamma * lax.rsqrt(var + BN_EPS)
    u = a * sc
    v = b_ * sc
    w0 = bn_beta - mu * sc

    eye4 = jnp.eye(4, dtype=f32)
    ones32r = jnp.ones((1, 32), f32)
    tile4 = lambda vec: jnp.tile(vec.reshape(1, H), (1, 4)).reshape(1, 128)
    vecs = jnp.concatenate([
        tile4(u), tile4(v), tile4(w0), tile4(b2), tile4(bx1),
        jnp.full((1, 128), bm[0], f32), tile4(Wm[0]),
        jnp.zeros((1, 128), f32),
    ], axis=0)
    mats = jnp.concatenate([
        jnp.kron(eye4, W2.T), jnp.kron(eye4, Wx1.T)], axis=0)
    e0 = jnp.zeros((H, 1), f32).at[0, 0].set(1.0)
    cols = jnp.concatenate([
        jnp.kron(eye4, Wx2.T),
        jnp.kron(eye4, jnp.ones((H, 1), f32)),
    ], axis=1)
    del e0
    r4 = jnp.kron(eye4, ones32r)

    n4 = n_psi.reshape(E // 4, 4)
    d4 = d_psi.reshape(E // 4, 4)
    m2, phi4 = _tc2(n4, d4, mats, cols, r4, vecs)
    m_ij = m2.reshape(E, H)
    phi = phi4.reshape(E)

    planes = _sc_b(dxp, dyp, dzp, phi, i_idx)

    NB = 2048
    nb = NPAD // NB
    x3 = jnp.pad(x, ((0, NPAD - N), (0, 0))).reshape(nb, NB, 3)
    pr = [p.reshape(nb, 1, NB) for p in planes]
    x_tilde = _tc3(x3, pr).reshape(NPAD, 3)[:N]
    return (x_tilde, m_ij)


# speedup vs baseline: 6.8801x; 1.1611x over previous
import functools

import jax
import jax.numpy as jnp
from jax import lax
from jax.experimental import pallas as pl
from jax.experimental.pallas import tpu as pltpu
from jax.experimental.pallas import tpu_sc as plsc

N = 50000
E = 1600000
D = 3
H = 32
BN_EPS = 1e-5

CH = 128
NCH = E // CH
NW = 32
BASE_T = NCH // NW
EXTRA_T = NCH % NW
NT2 = (BASE_T + 1) // 2 + 2
NPAD = 51200
PB = NPAD // 16

_G16 = CH // 16


def _psi(v):
    return jnp.sign(v) * jnp.log(jnp.abs(v) + 1.0)


def _sc_a_body(x0, x1, x2, i_hbm, j_hbm, nrm_o, dot_o, dx_o, dy_o, dz_o, *sc):
    wid = lax.axis_index("s") * 2 + lax.axis_index("c")
    nt = BASE_T + jnp.where(wid < EXTRA_T, 1, 0)
    ii = sc[0:2]
    jj = sc[2:4]
    gi = (sc[4:7], sc[7:10])
    gj = (sc[10:13], sc[13:16])
    st = (sc[16:21], sc[21:26])
    s_idx = sc[26:28]
    s_gat = sc[28:30]
    s_out = sc[30:32]
    tabs = (x0, x1, x2)
    outs = (nrm_o, dot_o, dx_o, dy_o, dz_o)

    def issue_gat(s):
        for k in range(3):
            pltpu.async_copy(tabs[k].at[ii[s]], gi[s][k], s_gat[s])
            pltpu.async_copy(tabs[k].at[jj[s]], gj[s][k], s_gat[s])

    def drain_gat(s):
        for k in range(3):
            pltpu.make_async_copy(x0.at[pl.ds(0, CH)], gi[s][k], s_gat[s]).wait()
            pltpu.make_async_copy(x0.at[pl.ds(0, CH)], gj[s][k], s_gat[s]).wait()

    def drain_out(s):
        for k in range(5):
            pltpu.make_async_copy(st[s][k], outs[k].at[pl.ds(0, CH)],
                                  s_out[s]).wait()

    def compute(s, off):
        for g in range(_G16):
            sl = pl.ds(g * 16, 16)
            a0, a1, a2 = gi[s][0][sl], gi[s][1][sl], gi[s][2][sl]
            b0, b1, b2 = gj[s][0][sl], gj[s][1][sl], gj[s][2][sl]
            dx = a0 - b0
            dy = a1 - b1
            dz = a2 - b2
            st[s][2][sl] = dx
            st[s][3][sl] = dy
            st[s][4][sl] = dz
            st[s][0][sl] = dx * dx + dy * dy + dz * dz
            st[s][1][sl] = a0 * b0 + a1 * b1 + a2 * b2
        for k in range(5):
            pltpu.async_copy(st[s][k], outs[k].at[pl.ds(off, CH)], s_out[s])

    off0 = wid * CH
    pltpu.sync_copy(i_hbm.at[pl.ds(off0, CH)], ii[0])
    pltpu.sync_copy(j_hbm.at[pl.ds(off0, CH)], jj[0])
    issue_gat(0)
    off1 = (wid + NW) * CH
    pltpu.async_copy(i_hbm.at[pl.ds(off1, CH)], ii[1], s_idx[1])
    pltpu.async_copy(j_hbm.at[pl.ds(off1, CH)], jj[1], s_idx[1])

    def it(t2, carry):
        for b in (0, 1):
            t = t2 * 2 + b

            @pl.when(t < nt)
            def _():
                drain_gat(b)

            @pl.when(jnp.logical_and(t >= 2, t - 2 < nt))
            def _():
                drain_out(b)

            @pl.when(t + 1 < nt)
            def _():
                pltpu.make_async_copy(i_hbm.at[pl.ds(0, CH)], ii[1 - b],
                                      s_idx[1 - b]).wait()
                pltpu.make_async_copy(i_hbm.at[pl.ds(0, CH)], jj[1 - b],
                                      s_idx[1 - b]).wait()
                issue_gat(1 - b)

            @pl.when(t + 2 < nt)
            def _():
                offn = (wid + (t + 2) * NW) * CH
                pltpu.async_copy(i_hbm.at[pl.ds(offn, CH)], ii[b], s_idx[b])
                pltpu.async_copy(j_hbm.at[pl.ds(offn, CH)], jj[b], s_idx[b])

            @pl.when(t < nt)
            def _():
                compute(b, (wid + t * NW) * CH)
        return carry

    lax.fori_loop(0, NT2, it, 0)


def _sc_a(x0, x1, x2, i_idx, j_idx):
    f32 = jnp.float32
    out = jax.ShapeDtypeStruct((E,), f32)
    mesh = plsc.VectorSubcoreMesh(core_axis_name="c", subcore_axis_name="s")
    return pl.kernel(
        _sc_a_body,
        out_type=[out] * 5,
        mesh=mesh,
        scratch_types=(
            [pltpu.VMEM((CH,), jnp.int32)] * 4
            + [pltpu.VMEM((CH,), f32)] * 22
            + [pltpu.SemaphoreType.DMA] * 6
        ),
    )(x0, x1, x2, i_idx, j_idx)


def _tc1_body(nr_ref, dr_ref, n_out, d_out, s_out):
    b = pl.program_id(0)
    rows = jnp.minimum(E - b * _T1B, _T1B) // 128
    msk = jax.lax.broadcasted_iota(jnp.int32, (_T1B // 128, 128), 0) < rows
    nr = jnp.where(msk, nr_ref[...].reshape(_T1B // 128, 128), 0.0)
    dr = jnp.where(msk, dr_ref[...].reshape(_T1B // 128, 128), 0.0)
    n = jnp.log(nr + 1.0)
    d = _psi(dr)
    n_out[...] = n.reshape(_T1B)
    d_out[...] = d.reshape(_T1B)

    @pl.when(b == 0)
    def _():
        for k in range(8):
            s_out[k] = 0.0

    s_out[0] += jnp.sum(n)
    s_out[1] += jnp.sum(d)
    s_out[2] += jnp.sum(n * n)
    s_out[3] += jnp.sum(d * d)
    s_out[4] += jnp.sum(n * d)


_T1B = 32768


def _tc1(nrm, dot):
    f32 = jnp.float32
    grid = ((E + _T1B - 1) // _T1B,)
    blk = pl.BlockSpec((_T1B,), lambda b: (b,))
    return pl.pallas_call(
        _tc1_body,
        grid=grid,
        in_specs=[blk, blk],
        out_specs=[blk, blk,
                   pl.BlockSpec(memory_space=pltpu.SMEM)],
        out_shape=[jax.ShapeDtypeStruct((E,), f32),
                   jax.ShapeDtypeStruct((E,), f32),
                   jax.ShapeDtypeStruct((8,), f32)],
    )(nrm, dot)


_RB = 512


def _tc2_body(n1_ref, d1_ref, mats_ref, cols_ref, r4_ref, vecs_ref,
              a512_ref, c128_ref, m2_ref, phi4_ref):
    f32 = jnp.float32
    b = pl.program_id(0)
    vcnt = jnp.minimum(E - b * 4 * _RB, 4 * _RB)
    vmsk = (jax.lax.broadcasted_iota(jnp.int32, (16, 128), 0) * 128 +
            jax.lax.broadcasted_iota(jnp.int32, (16, 128), 1)) < vcnt
    r512 = (jax.lax.broadcasted_iota(jnp.int32, (_RB, 16), 0) // 32 ==
            jax.lax.broadcasted_iota(jnp.int32, (_RB, 16), 1)).astype(f32)
    a512 = a512_ref[...]
    c128 = c128_ref[...]

    def rep(v1d):
        v16 = jnp.where(vmsk, v1d.reshape(16, 128), 0.0)
        vr = jnp.dot(r512, v16, preferred_element_type=f32)
        return jnp.dot(a512 * vr, c128, preferred_element_type=f32)

    nr = rep(n1_ref[...])
    dr = rep(d1_ref[...])
    r4 = r4_ref[...]
    u = vecs_ref[0:1, :]
    v = vecs_ref[1:2, :]
    w = vecs_ref[2:3, :]
    h = jnp.maximum(nr * u + dr * v + w, 0.0)
    m1 = jnp.dot(h, mats_ref[0:128, :], preferred_element_type=f32)
    m1 = jnp.maximum(m1 + vecs_ref[3:4, :], 0.0)
    g4 = jnp.dot(m1 * vecs_ref[6:7, :], cols_ref[:, 4:8],
                 preferred_element_type=f32)
    g4 = 1.0 / (1.0 + jnp.exp(-(g4 + vecs_ref[5:6, 0:4])))
    gate = jnp.dot(g4, r4, preferred_element_type=f32)
    mij = m1 * gate
    t = jnp.dot(mij, mats_ref[128:256, :], preferred_element_type=f32)
    t = jnp.maximum(t + vecs_ref[4:5, :], 0.0)
    phi4 = jnp.dot(t, cols_ref[:, 0:4], preferred_element_type=f32)
    m2_ref[...] = mij
    phi4_ref[...] = phi4


def _tc2(n1, d1, mats, cols, r4, vecs, a512, c128):
    f32 = jnp.float32
    E4 = E // 4
    grid = ((E4 + _RB - 1) // _RB,)
    full = lambda shp: pl.BlockSpec(shp, lambda b: tuple(0 for _ in shp))
    return pl.pallas_call(
        _tc2_body,
        grid=grid,
        in_specs=[pl.BlockSpec((4 * _RB,), lambda b: (b,)),
                  pl.BlockSpec((4 * _RB,), lambda b: (b,)),
                  full((256, 128)), full((128, 8)), full((4, 128)),
                  full((8, 128)), full((_RB, 128)), full((128, 128))],
        out_specs=[pl.BlockSpec((_RB, 128), lambda b: (b, 0)),
                   pl.BlockSpec((_RB, 4), lambda b: (b, 0))],
        out_shape=[jax.ShapeDtypeStruct((E4, 128), f32),
                   jax.ShapeDtypeStruct((E4, 4), f32)],
    )(n1, d1, mats, cols, r4, vecs, a512, c128)


def _sc_b_body(dx_p, dy_p, dz_p, phi_p, i_hbm,
               p0x, p0y, p0z, p0c, p1x, p1y, p1z, p1c, *sc):
    cid = lax.axis_index("c")
    sid = lax.axis_index("s")
    wid = sid * 2 + cid
    nt = BASE_T + jnp.where(wid < EXTRA_T, 1, 0)
    f32 = jnp.float32
    ld = (sc[0:5], sc[5:10])
    on_b = sc[10]
    zbuf = sc[11]
    accs = sc[12:16]
    s_ld = sc[16:18]
    srcs = (i_hbm, dx_p, dy_p, dz_p, phi_p)
    pls = ((p0x, p0y, p0z, p0c), (p1x, p1y, p1z, p1c))

    for g in range(_G16):
        on_b[pl.ds(g * 16, 16)] = jnp.ones((16,), f32)

    def zst(k, c):
        zbuf[pl.ds(k * 16, 16)] = jnp.zeros((16,), f32)
        return c

    lax.fori_loop(0, PB // 16, zst, 0)

    for a in accs:
        pltpu.sync_copy(zbuf, a.at[pl.ds(sid * PB, PB)])

    plsc.subcore_barrier()

    for sref, dbuf in zip(srcs, ld[0]):
        pltpu.sync_copy(sref.at[pl.ds(wid * CH, CH)], dbuf)
    for sref, dbuf in zip(srcs, ld[1]):
        pltpu.async_copy(sref.at[pl.ds((wid + NW) * CH, CH)], dbuf, s_ld[1])

    def it(t2, carry):
        for b in (0, 1):
            t = t2 * 2 + b

            @pl.when(jnp.logical_and(t >= 1, t < nt))
            def _():
                for sref, dbuf in zip(srcs, ld[b]):
                    pltpu.make_async_copy(sref.at[pl.ds(0, CH)], dbuf,
                                          s_ld[b]).wait()

            @pl.when(t < nt)
            def _():
                iv, bx, by, bz, bp = ld[b]
                for g in range(_G16):
                    sl = pl.ds(g * 16, 16)
                    ph = bp[sl]
                    bx[sl] = jnp.clip(bx[sl] * ph, -100.0, 100.0)
                    by[sl] = jnp.clip(by[sl] * ph, -100.0, 100.0)
                    bz[sl] = jnp.clip(bz[sl] * ph, -100.0, 100.0)
                pltpu.sync_copy(bx, accs[0].at[iv], add=True)
                pltpu.sync_copy(by, accs[1].at[iv], add=True)
                pltpu.sync_copy(bz, accs[2].at[iv], add=True)
                pltpu.sync_copy(on_b, accs[3].at[iv], add=True)

            @pl.when(t + 2 < nt)
            def _():
                offn = (wid + (t + 2) * NW) * CH
                for sref, dbuf in zip(srcs, ld[b]):
                    pltpu.async_copy(sref.at[pl.ds(offn, CH)], dbuf, s_ld[b])
        return carry

    lax.fori_loop(0, NT2, it, 0)
    plsc.subcore_barrier()

    for ci in (0, 1):
        @pl.when(cid == ci)
        def _(ci=ci):
            for a, p in zip(accs, pls[ci]):
                pltpu.sync_copy(a.at[pl.ds(sid * PB, PB)],
                                p.at[pl.ds(sid * PB, PB)])


def _sc_b(dx_p, dy_p, dz_p, phi_p, i_idx):
    f32 = jnp.float32
    mesh = plsc.VectorSubcoreMesh(core_axis_name="c", subcore_axis_name="s")
    return pl.kernel(
        _sc_b_body,
        out_type=[jax.ShapeDtypeStruct((NPAD,), f32)] * 8,
        mesh=mesh,
        scratch_types=(
            ([pltpu.VMEM((CH,), jnp.int32)] + [pltpu.VMEM((CH,), f32)] * 4) * 2
            + [pltpu.VMEM((CH,), f32)]
            + [pltpu.VMEM((PB,), f32)]
            + [pltpu.VMEM_SHARED((NPAD,), f32)] * 4
            + [pltpu.SemaphoreType.DMA] * 2
        ),
    )(dx_p, dy_p, dz_p, phi_p, i_idx)


_NB3 = 2048


def _tc3_body(x0r, x1r, x2r, p0x, p0y, p0z, p0c, p1x, p1y, p1z, p1c,
              o0, o1, o2):
    R = _NB3 // 128
    cnt = jnp.maximum(p0c[...].reshape(R, 128) + p1c[...].reshape(R, 128),
                      1.0)
    sx = (p0x[...].reshape(R, 128) + p1x[...].reshape(R, 128)) / cnt
    sy = (p0y[...].reshape(R, 128) + p1y[...].reshape(R, 128)) / cnt
    sz = (p0z[...].reshape(R, 128) + p1z[...].reshape(R, 128)) / cnt
    o0[...] = (x0r[...].reshape(R, 128) + sx).reshape(_NB3)
    o1[...] = (x1r[...].reshape(R, 128) + sy).reshape(_NB3)
    o2[...] = (x2r[...].reshape(R, 128) + sz).reshape(_NB3)


def _tc3(xt, planes):
    f32 = jnp.float32
    grid = (NPAD // _NB3,)
    nblk = pl.BlockSpec((_NB3,), lambda b: (b,))
    return pl.pallas_call(
        _tc3_body,
        grid=grid,
        in_specs=[nblk] * 3 + [nblk] * 8,
        out_specs=[nblk] * 3,
        out_shape=[jax.ShapeDtypeStruct((NPAD,), f32)] * 3,
    )(xt[0], xt[1], xt[2], *planes)


def kernel(x, edge_index, W1, bn_gamma, bn_beta, W2, b2, Wx1, bx1, Wx2, Wm, bm):
    f32 = jnp.float32
    i_idx = edge_index[0]
    j_idx = edge_index[1]
    xt = x.T
    nrm_raw, dot_raw, dxp, dyp, dzp = _sc_a(xt[0], xt[1], xt[2], i_idx, j_idx)

    n_psi, d_psi, sums = _tc1(nrm_raw, dot_raw)

    En, Ed, En2, Ed2, End = (sums[k] / E for k in range(5))
    a = W1[:, 0]
    b_ = W1[:, 1]
    mu = a * En + b_ * Ed
    eh2 = a * a * En2 + 2.0 * a * b_ * End + b_ * b_ * Ed2
    var = eh2 - mu * mu
    sc = bn_gamma * lax.rsqrt(var + BN_EPS)
    u = a * sc
    v = b_ * sc
    w0 = bn_beta - mu * sc

    eye4 = jnp.eye(4, dtype=f32)
    la = jnp.arange(128)
    a512 = (la[None, :] // 4 ==
            (jnp.arange(_RB) % 32)[:, None]).astype(f32)
    c128 = (la[:, None] % 4 == la[None, :] // 32).astype(f32)
    ones32r = jnp.ones((1, 32), f32)
    tile4 = lambda vec: jnp.tile(vec.reshape(1, H), (1, 4)).reshape(1, 128)
    vecs = jnp.concatenate([
        tile4(u), tile4(v), tile4(w0), tile4(b2), tile4(bx1),
        jnp.full((1, 128), bm[0], f32), tile4(Wm[0]),
        jnp.zeros((1, 128), f32),
    ], axis=0)
    mats = jnp.concatenate([
        jnp.kron(eye4, W2.T), jnp.kron(eye4, Wx1.T)], axis=0)
    e0 = jnp.zeros((H, 1), f32).at[0, 0].set(1.0)
    cols = jnp.concatenate([
        jnp.kron(eye4, Wx2.T),
        jnp.kron(eye4, jnp.ones((H, 1), f32)),
    ], axis=1)
    del e0
    r4 = jnp.kron(eye4, ones32r)

    m2, phi4 = _tc2(n_psi, d_psi, mats, cols, r4, vecs, a512, c128)
    m_ij = m2.reshape(E, H)
    phi = phi4.reshape(E)

    planes = _sc_b(dxp, dyp, dzp, phi, i_idx)

    xtp = jnp.pad(xt, ((0, 0), (0, NPAD - N)))
    o0, o1, o2 = _tc3(xtp, planes)
    x_tilde = jnp.stack([o0[:N], o1[:N], o2[:N]], axis=1)
    return (x_tilde, m_ij)
